# baseline (speedup 1.0000x reference)
import functools

import jax
import jax.numpy as jnp
from jax import lax
from jax.experimental import pallas as pl
from jax.experimental.pallas import tpu as pltpu
from jax.experimental.pallas import tpu_sc as plsc

BATCH = 16384
DIM = 64
LANES = 16
NUM_WORKERS = 32
B_PER_W = BATCH // NUM_WORKERS
CHUNK = 128
N_CHUNKS = B_PER_W // CHUNK
GROUPS = B_PER_W // LANES


def _body(ent_hbm, rel_hbm, hidx_hbm, ridx_hbm, tidx_hbm, out_hbm,
          hidx_v, ridx_v, tidx_v, hrows_v, rrows_v, trows_v, tsp_v, out_v,
          sem):
    wid = lax.axis_index("s") * 2 + lax.axis_index("c")
    row0 = wid * N_CHUNKS
    base = wid * B_PER_W

    pltpu.sync_copy(hidx_hbm.at[pl.ds(row0, N_CHUNKS)], hidx_v)
    pltpu.sync_copy(ridx_hbm.at[pl.ds(row0, N_CHUNKS)], ridx_v)
    pltpu.sync_copy(tidx_hbm.at[pl.ds(row0, N_CHUNKS)], tidx_v)

    copies = []
    for k in range(N_CHUNKS):
        dst = pl.ds(k * CHUNK, CHUNK)
        copies.append(pltpu.async_copy(
            ent_hbm.at[hidx_v.at[k]], hrows_v.at[dst], sem))
        copies.append(pltpu.async_copy(
            rel_hbm.at[ridx_v.at[k]], rrows_v.at[dst], sem))
        copies.append(pltpu.async_copy(
            ent_hbm.at[tidx_v.at[k]], trows_v.at[dst], sem))
    for c in copies:
        c.wait()

    lanes = lax.iota(jnp.int32, LANES)
    zero = jnp.zeros((LANES,), jnp.float32)
    half = jnp.full((LANES,), 0.5, jnp.float32)
    three_half = jnp.full((LANES,), 1.5, jnp.float32)
    magic = jnp.full((LANES,), 0x5F3759DF, jnp.int32)

    def group(g, _):
        tot = zero
        for t in range(LANES):
            i = g * LANES + t
            acc = zero
            for j in range(DIM // LANES):
                sl = pl.ds(j * LANES, LANES)
                d = hrows_v[i, sl] + rrows_v[i, sl] - trows_v[i, sl]
                acc = acc + d * d
            s = acc[0]
            for c in range(1, LANES):
                s = s + acc[c]
            tot = jnp.where(lanes == t, s, tot)
        xi = lax.bitcast_convert_type(tot, jnp.int32)
        y = lax.bitcast_convert_type(
            magic - lax.shift_right_logical(xi, 1), jnp.float32)
        hx = half * tot
        for _ in range(3):
            y = y * (three_half - hx * y * y)
        out_v[pl.ds(g * LANES, LANES)] = tot * y
        return 0

    lax.fori_loop(0, GROUPS, group, 0)
    pltpu.sync_copy(out_v, out_hbm.at[pl.ds(base, B_PER_W)])


@jax.jit
def kernel(triples, entity_table, relation_table):
    hidx = triples[:, 0].reshape(BATCH // CHUNK, CHUNK)
    ridx = triples[:, 1].reshape(BATCH // CHUNK, CHUNK)
    tidx = triples[:, 2].reshape(BATCH // CHUNK, CHUNK)

    run = functools.partial(
        pl.kernel,
        out_type=jax.ShapeDtypeStruct((BATCH,), jnp.float32),
        mesh=plsc.VectorSubcoreMesh(core_axis_name="c", subcore_axis_name="s"),
        compiler_params=pltpu.CompilerParams(use_tc_tiling_on_sc=False),
        scratch_types=[
            pltpu.VMEM((N_CHUNKS, CHUNK), jnp.int32),
            pltpu.VMEM((N_CHUNKS, CHUNK), jnp.int32),
            pltpu.VMEM((N_CHUNKS, CHUNK), jnp.int32),
            pltpu.VMEM((B_PER_W, DIM), jnp.float32),
            pltpu.VMEM((B_PER_W, DIM), jnp.float32),
            pltpu.VMEM((B_PER_W, DIM), jnp.float32),
            pltpu.VMEM((LANES,), jnp.float32),
            pltpu.VMEM((B_PER_W,), jnp.float32),
            pltpu.SemaphoreType.DMA,
        ],
    )(_body)
    return run(entity_table, relation_table, hidx, ridx, tidx)
